# Initial kernel scaffold; baseline (speedup 1.0000x reference)
import jax
import jax.numpy as jnp
from jax.experimental import pallas as pl


def kernel(x, W1, b1, W2, b2, kge_emb):
    raise NotImplementedError("write your pallas kernel here")


# speedup vs baseline: 5.4445x; 5.4445x over previous
import functools

import jax
import jax.numpy as jnp
from jax import lax
from jax.experimental import pallas as pl
from jax.experimental.pallas import tpu as pltpu
from jax.experimental.pallas import tpu_sc as plsc

_B = 1024
_DM = 2048
_KE = 100000
_DK = 256
_K = 64

_ETILE = 2048
_NTILE = 49
_NPAD = _NTILE * _ETILE
_CW = 128
_NSUB = _NPAD // _CW
_SUB_PER_TILE = _ETILE // _CW
_NEG = -3.0e38
_BIGI = 2**30

_MLP_RB = 256
_SEL_RB = 128
_CMB_RB = 128


def _mlp_body(x_ref, w1_ref, b1_ref, w2_ref, b2_ref, o_ref):
    h = jnp.dot(x_ref[...], w1_ref[...], preferred_element_type=jnp.float32)
    h = jnp.maximum(h + b1_ref[...], 0.0)
    o_ref[...] = (
        jnp.dot(h, w2_ref[...], preferred_element_type=jnp.float32) + b2_ref[...]
    )


def _mlp(x, W1, b1, W2, b2):
    return pl.pallas_call(
        _mlp_body,
        grid=(_B // _MLP_RB,),
        in_specs=[
            pl.BlockSpec((_MLP_RB, _DM), lambda i: (i, 0)),
            pl.BlockSpec((_DM, _DM), lambda i: (0, 0)),
            pl.BlockSpec((1, _DM), lambda i: (0, 0)),
            pl.BlockSpec((_DM, _DK), lambda i: (0, 0)),
            pl.BlockSpec((1, _DK), lambda i: (0, 0)),
        ],
        out_specs=pl.BlockSpec((_MLP_RB, _DK), lambda i: (i, 0)),
        out_shape=jax.ShapeDtypeStruct((_B, _DK), jnp.float32),
    )(x, W1, b1.reshape(1, _DM), W2, b2.reshape(1, _DK))


def _dot_body(sp_ref, kge_ref, d_ref, t_ref, m_scr):
    i = pl.program_id(0)
    dt = lax.dot_general(
        sp_ref[...], kge_ref[...], (((1,), (1,)), ((), ())),
        preferred_element_type=jnp.float32,
    )
    col0 = i * _ETILE
    cols = col0 + lax.broadcasted_iota(jnp.int32, (_B, _ETILE), 1)
    dt = jnp.where(cols < _KE, dt, _NEG)
    dt3 = dt.reshape(_B, _SUB_PER_TILE, _CW)
    d_ref[...] = dt3
    m = jnp.max(dt3, axis=2)
    m_scr[pl.ds(i * _SUB_PER_TILE, _SUB_PER_TILE), :] = m.T

    @pl.when(i == _NTILE - 1)
    def _():
        sub_iota = lax.broadcasted_iota(jnp.int32, (_NSUB, _B), 0)
        row_iota = lax.broadcasted_iota(jnp.int32, (_K, _B), 0)

        def body(k, carry):
            m_cur, t_acc = carry
            mx = jnp.max(m_cur, axis=0, keepdims=True)
            sel = jnp.where(m_cur == mx, sub_iota, _BIGI)
            si = jnp.min(sel, axis=0, keepdims=True)
            t_acc = jnp.where(row_iota == k, si, t_acc)
            m_cur = jnp.where(sub_iota == si, _NEG, m_cur)
            return m_cur, t_acc

        t0 = jnp.zeros((_K, _B), jnp.int32)
        _, t_final = lax.fori_loop(0, _K, body, (m_scr[...], t0))
        t_ref[...] = t_final


def _dot_topchunks(spanned, kge_emb):
    return pl.pallas_call(
        _dot_body,
        grid=(_NTILE,),
        in_specs=[
            pl.BlockSpec((_B, _DK), lambda i: (0, 0)),
            pl.BlockSpec((_ETILE, _DK), lambda i: (i, 0)),
        ],
        out_specs=[
            pl.BlockSpec((_B, _SUB_PER_TILE, _CW), lambda i: (0, i, 0)),
            pl.BlockSpec((_K, _B), lambda i: (0, 0)),
        ],
        out_shape=[
            jax.ShapeDtypeStruct((_B, _NSUB, _CW), jnp.float32),
            jax.ShapeDtypeStruct((_K, _B), jnp.int32),
        ],
        scratch_shapes=[pltpu.VMEM((_NSUB, _B), jnp.float32)],
    )(spanned, kge_emb)


_NC, _NS = 2, 16
_NW = _NC * _NS


@functools.cache
def _make_sc_gather(D, N, chunk):
    per_w = N // _NW
    n_chunks = per_w // chunk
    mesh = plsc.VectorSubcoreMesh(
        core_axis_name="c", subcore_axis_name="s",
        num_cores=_NC, num_subcores=_NS,
    )

    @functools.partial(
        pl.kernel,
        mesh=mesh,
        out_type=jax.ShapeDtypeStruct((N, D), jnp.float32),
        scratch_types=[
            pltpu.VMEM((chunk,), jnp.int32),
            pltpu.VMEM((chunk, D), jnp.float32),
            pltpu.SemaphoreType.DMA,
        ],
    )
    def gather_k(table_hbm, idx_hbm, out_hbm, idx_v, rows_v, sem):
        wid = lax.axis_index("s") * _NC + lax.axis_index("c")

        def step(ci, carry):
            b0 = wid * per_w + ci * chunk
            pltpu.sync_copy(idx_hbm.at[pl.ds(b0, chunk)], idx_v)
            pltpu.async_copy(table_hbm.at[idx_v], rows_v, sem).wait()
            pltpu.sync_copy(rows_v, out_hbm.at[pl.ds(b0, chunk)])
            return carry

        lax.fori_loop(0, n_chunks, step, 0)

    return gather_k


def _gather_chunks(table, idx):
    return _make_sc_gather(_CW, _B * _K, 128)(table, idx)


def _gather_emb(table, idx):
    return _make_sc_gather(_DK, _B * _K, 128)(table, idx)


def _sel_body(cand_ref, t_ref, w_ref, g_ref):
    c = cand_ref[...].reshape(_SEL_RB, _K, _CW)
    pos = (
        lax.broadcasted_iota(jnp.int32, (_SEL_RB, _K, _CW), 1) * _CW
        + lax.broadcasted_iota(jnp.int32, (_SEL_RB, _K, _CW), 2)
    )
    lane_k = lax.broadcasted_iota(jnp.int32, (_SEL_RB, _K), 1)

    def body(k, carry):
        c_cur, vals, idxs = carry
        m1 = jnp.max(c_cur, axis=2)
        mx = jnp.max(m1, axis=1, keepdims=True)
        sel = jnp.where(c_cur == mx[:, :, None], pos, _BIGI)
        s1 = jnp.min(sel, axis=2)
        si = jnp.min(s1, axis=1, keepdims=True)
        vals = jnp.where(lane_k == k, mx, vals)
        idxs = jnp.where(lane_k == k, si, idxs)
        c_cur = jnp.where(pos == si[:, :, None], _NEG, c_cur)
        return c_cur, vals, idxs

    vals0 = jnp.full((_SEL_RB, _K), _NEG, jnp.float32)
    idxs0 = jnp.zeros((_SEL_RB, _K), jnp.int32)
    _, vals, idxs = lax.fori_loop(0, _K, body, (c, vals0, idxs0))

    jsel = idxs // _CW
    off = idxs % _CW
    t = t_ref[...]
    oh = jsel[:, :, None] == lax.broadcasted_iota(jnp.int32, (_SEL_RB, _K, _K), 2)
    tsub = jnp.sum(jnp.where(oh, t[:, None, :], 0), axis=2)
    g_ref[...] = tsub * _CW + off

    u = jnp.exp(vals)
    wv = jnp.exp(u - u[:, 0:1])
    w_ref[...] = wv / jnp.sum(wv, axis=1, keepdims=True)


def _select(cand, T):
    return pl.pallas_call(
        _sel_body,
        grid=(_B // _SEL_RB,),
        in_specs=[
            pl.BlockSpec((_SEL_RB * _K, _CW), lambda i: (i, 0)),
            pl.BlockSpec((_SEL_RB, _K), lambda i: (i, 0)),
        ],
        out_specs=[
            pl.BlockSpec((_SEL_RB, _K), lambda i: (i, 0)),
            pl.BlockSpec((_SEL_RB, _K), lambda i: (i, 0)),
        ],
        out_shape=[
            jax.ShapeDtypeStruct((_B, _K), jnp.float32),
            jax.ShapeDtypeStruct((_B, _K), jnp.int32),
        ],
    )(cand, T)


def _cmb_body(rows_ref, w_ref, o_ref):
    rv = rows_ref[...].reshape(_CMB_RB, _K, _DK)
    o_ref[...] = jnp.sum(rv * w_ref[...][:, :, None], axis=1)


def _combine(rows, w):
    return pl.pallas_call(
        _cmb_body,
        grid=(_B // _CMB_RB,),
        in_specs=[
            pl.BlockSpec((_CMB_RB * _K, _DK), lambda i: (i, 0)),
            pl.BlockSpec((_CMB_RB, _K), lambda i: (i, 0)),
        ],
        out_specs=pl.BlockSpec((_CMB_RB, _DK), lambda i: (i, 0)),
        out_shape=jax.ShapeDtypeStruct((_B, _DK), jnp.float32),
    )(rows, w)


def kernel(x, W1, b1, W2, b2, kge_emb):
    spanned = _mlp(x, W1, b1, W2, b2)
    d3, t_t = _dot_topchunks(spanned, kge_emb)
    T = t_t.T
    chunk_idx = (
        jnp.arange(_B, dtype=jnp.int32)[:, None] * _NSUB + T
    ).reshape(-1)
    cand = _gather_chunks(d3.reshape(_B * _NSUB, _CW), chunk_idx)
    w, gidx = _select(cand, T)
    rows = _gather_emb(kge_emb, gidx.reshape(-1))
    return _combine(rows, w)
